# baseline (speedup 1.0000x reference)
import jax
import jax.numpy as jnp
from jax import lax
from jax.experimental import pallas as pl
from jax.experimental.pallas import tpu as pltpu
from jax.experimental.pallas import tpu_sc as plsc

N_IN = 10000
D = 256
E = 513
E_PAD = 1024
N_OUT = N_IN + E
NC, NS = 2, 16
NW = NC * NS
EPW = E_PAD // NW
LANES = 16
CP_ROWS = 1000


def _tail_kernel(x_hbm, i0_hbm, i1_hbm, t_hbm,
                 idx0_v, idx1_v, buf0, buf1, sem0, sem1):
    c = lax.axis_index("c")
    s = lax.axis_index("s")
    wid = s * NC + c
    ebase = wid * EPW
    pltpu.sync_copy(i0_hbm.at[pl.ds(ebase, EPW)], idx0_v)
    pltpu.sync_copy(i1_hbm.at[pl.ds(ebase, EPW)], idx1_v)
    hg0 = pltpu.async_copy(x_hbm.at[idx0_v], buf0, sem0)
    hg1 = pltpu.async_copy(x_hbm.at[idx1_v], buf1, sem1)
    hg0.wait()
    hg1.wait()

    def body(e, carry):
        for j in range(D // LANES):
            sl = pl.ds(j * LANES, LANES)
            buf0[e, sl] = (buf0[e, sl] + buf1[e, sl]) * 0.5
        return carry

    lax.fori_loop(0, EPW, body, 0)

    pltpu.sync_copy(buf0, t_hbm.at[pl.ds(ebase, EPW)])


def _copy_body(x_ref, o_ref):
    o_ref[...] = x_ref[...]


def _insert_body(o0_ref, t_ref, o_ref):
    o_ref[...] = t_ref[...]


@jax.jit
def _run(x, idx0, idx1):
    mesh = plsc.VectorSubcoreMesh(core_axis_name="c", subcore_axis_name="s",
                                  num_cores=NC, num_subcores=NS)
    t = pl.kernel(
        _tail_kernel,
        out_type=jax.ShapeDtypeStruct((E_PAD, D), jnp.float32),
        mesh=mesh,
        scratch_types=[
            pltpu.VMEM((EPW,), jnp.int32),
            pltpu.VMEM((EPW,), jnp.int32),
            pltpu.VMEM((EPW, D), jnp.float32),
            pltpu.VMEM((EPW, D), jnp.float32),
            pltpu.SemaphoreType.DMA,
            pltpu.SemaphoreType.DMA,
        ],
    )(x, idx0, idx1)

    out0 = pl.pallas_call(
        _copy_body,
        grid=(N_IN // CP_ROWS,),
        in_specs=[pl.BlockSpec((CP_ROWS, D), lambda i: (i, 0))],
        out_specs=pl.BlockSpec((CP_ROWS, D), lambda i: (i, 0)),
        out_shape=jax.ShapeDtypeStruct((N_OUT, D), jnp.float32),
    )(x)

    out = pl.pallas_call(
        _insert_body,
        grid=(1,),
        in_specs=[
            pl.BlockSpec((8, 128), lambda i: (0, 0)),
            pl.BlockSpec((CP_ROWS, D), lambda i: (0, 0)),
        ],
        out_specs=pl.BlockSpec((CP_ROWS, D), lambda i: (N_IN // CP_ROWS, 0)),
        out_shape=jax.ShapeDtypeStruct((N_OUT, D), jnp.float32),
        input_output_aliases={0: 0},
    )(out0, t)
    return out


def kernel(input, pool_idx):
    idx = pool_idx.astype(jnp.int32)
    pad = jnp.broadcast_to(idx[-1:], (E_PAD - E, 2))
    idx = jnp.concatenate([idx, pad], axis=0)
    return _run(input, idx[:, 0], idx[:, 1])


# speedup vs baseline: 1.8207x; 1.8207x over previous
import jax
import jax.numpy as jnp
from jax import lax
from jax.experimental import pallas as pl
from jax.experimental.pallas import tpu as pltpu
from jax.experimental.pallas import tpu_sc as plsc

N_IN = 10000
D = 256
E = 513
E_PAD = 520
T_ROWS = 1024
N_OUT = N_IN + E
NC, NS = 2, 16
NW = NC * NS
EPW = 16
EPC = 24
LANES = 16
CP_ROWS = 1000
CP_N = N_IN // CP_ROWS


def _tail_kernel(x_hbm, i0_hbm, i1_hbm, t_hbm,
                 idx0_v, idx1_v, buf0, buf1, sem0, sem1):
    c = lax.axis_index("c")
    s = lax.axis_index("s")
    wid = s * NC + c
    ebase = wid * EPW
    pltpu.sync_copy(i0_hbm.at[pl.ds(ebase, EPC)], idx0_v)
    pltpu.sync_copy(i1_hbm.at[pl.ds(ebase, EPC)], idx1_v)
    hg0 = pltpu.async_copy(x_hbm.at[idx0_v], buf0, sem0)
    hg1 = pltpu.async_copy(x_hbm.at[idx1_v], buf1, sem1)
    hg0.wait()
    hg1.wait()

    def body(e, carry):
        for j in range(D // LANES):
            sl = pl.ds(j * LANES, LANES)
            buf0[e, sl] = (buf0[e, sl] + buf1[e, sl]) * 0.5
        return carry

    lax.fori_loop(0, EPC, body, 0)

    pltpu.sync_copy(buf0, t_hbm.at[pl.ds(ebase, EPC)])


def _copy_body(x_hbm, o_hbm, bufs, rsems, wsems):
    hin = [pltpu.make_async_copy(x_hbm.at[pl.ds(c * CP_ROWS, CP_ROWS)],
                                 bufs[c], rsems[c])
           for c in range(CP_N)]
    for h in hin:
        h.start()
    hout = []
    for c in range(CP_N):
        hin[c].wait()
        h = pltpu.make_async_copy(bufs[c],
                                  o_hbm.at[pl.ds(c * CP_ROWS, CP_ROWS)],
                                  wsems[c])
        h.start()
        hout.append(h)
    for h in hout:
        h.wait()


def _insert_body(o0_ref, t_ref, o_ref):
    o_ref[...] = t_ref[...]


@jax.jit
def _run(x, idx0, idx1):
    mesh = plsc.VectorSubcoreMesh(core_axis_name="c", subcore_axis_name="s",
                                  num_cores=NC, num_subcores=NS)
    t = pl.kernel(
        _tail_kernel,
        out_type=jax.ShapeDtypeStruct((T_ROWS, D), jnp.float32),
        mesh=mesh,
        scratch_types=[
            pltpu.VMEM((EPC,), jnp.int32),
            pltpu.VMEM((EPC,), jnp.int32),
            pltpu.VMEM((EPC, D), jnp.float32),
            pltpu.VMEM((EPC, D), jnp.float32),
            pltpu.SemaphoreType.DMA,
            pltpu.SemaphoreType.DMA,
        ],
    )(x, idx0, idx1)

    out0 = pl.pallas_call(
        _copy_body,
        in_specs=[pl.BlockSpec(memory_space=pl.ANY)],
        out_specs=pl.BlockSpec(memory_space=pl.ANY),
        out_shape=jax.ShapeDtypeStruct((N_OUT, D), jnp.float32),
        scratch_shapes=[
            [pltpu.VMEM((CP_ROWS, D), jnp.float32) for _ in range(CP_N)],
            [pltpu.SemaphoreType.DMA for _ in range(CP_N)],
            [pltpu.SemaphoreType.DMA for _ in range(CP_N)],
        ],
    )(x)

    out = pl.pallas_call(
        _insert_body,
        grid=(1,),
        in_specs=[
            pl.BlockSpec((8, 128), lambda i: (0, 0)),
            pl.BlockSpec((CP_ROWS, D), lambda i: (0, 0)),
        ],
        out_specs=pl.BlockSpec((CP_ROWS, D), lambda i: (N_IN // CP_ROWS, 0)),
        out_shape=jax.ShapeDtypeStruct((N_OUT, D), jnp.float32),
        input_output_aliases={0: 0},
    )(out0, t)
    return out


def kernel(input, pool_idx):
    idx = pool_idx.astype(jnp.int32)
    pad = jnp.broadcast_to(idx[-1:], (E_PAD - E, 2))
    idx = jnp.concatenate([idx, pad], axis=0)
    return _run(input, idx[:, 0], idx[:, 1])
